# Initial kernel scaffold; baseline (speedup 1.0000x reference)
import jax
import jax.numpy as jnp
from jax.experimental import pallas as pl


def kernel(x, edge_index, params):
    raise NotImplementedError("write your pallas kernel here")


# speedup vs baseline: 6.6089x; 6.6089x over previous
import functools

import jax
import jax.numpy as jnp
from jax import lax
from jax.experimental import pallas as pl
from jax.experimental.pallas import tpu as pltpu
from jax.experimental.pallas import tpu_sc as plsc

_N = 10000
_E = 320000
_NP = 10240
_NC = 2
_NS = 16
_L = 16
_CHUNK = 128
_K = 79
_EP = _NC * _NS * _K * _CHUNK
_RPT = _NP // _NS
_ZR = 64


def _make_segsum(D):
    mesh = plsc.VectorSubcoreMesh(core_axis_name="c", subcore_axis_name="s")
    zsteps = _ZR * (D // _L)

    def body(z_hbm, srcg, dstg, out_hbm, src_v, dst_v, gbuf, zbuf, acc, sem):
        c = lax.axis_index("c")
        s = lax.axis_index("s")
        w = c * _NS + s

        def zb(i, carry):
            r = i // (D // _L)
            col = (i % (D // _L)) * _L
            zbuf[r, pl.ds(col, _L)] = jnp.zeros((_L,), jnp.float32)
            return carry

        lax.fori_loop(0, zsteps, zb, 0)

        def zacc(i, carry):
            pltpu.sync_copy(zbuf, acc.at[pl.ds(s * _RPT + i * _ZR, _ZR)])
            return carry

        lax.fori_loop(0, _RPT // _ZR, zacc, 0)

        pltpu.sync_copy(srcg.at[w], src_v)
        pltpu.sync_copy(dstg.at[w], dst_v)
        plsc.subcore_barrier()

        def step(j, carry):
            pltpu.async_copy(z_hbm.at[src_v.at[j]], gbuf, sem).wait()
            pltpu.sync_copy(gbuf, acc.at[dst_v.at[j]], add=True)
            return carry

        lax.fori_loop(0, _K, step, 0)

        plsc.subcore_barrier()
        pltpu.sync_copy(acc.at[pl.ds(s * _RPT, _RPT)],
                        out_hbm.at[c, pl.ds(s * _RPT, _RPT)])

    return pl.kernel(
        body,
        out_type=jax.ShapeDtypeStruct((_NC, _NP, D), jnp.float32),
        mesh=mesh,
        scratch_types=[
            pltpu.VMEM((_K, _CHUNK), jnp.int32),
            pltpu.VMEM((_K, _CHUNK), jnp.int32),
            pltpu.VMEM((_CHUNK, D), jnp.float32),
            pltpu.VMEM((_ZR, D), jnp.float32),
            pltpu.VMEM_SHARED((_NP, D), jnp.float32),
            pltpu.SemaphoreType.DMA,
        ],
        compiler_params=pltpu.CompilerParams(use_tc_tiling_on_sc=False),
    )


_segsum16 = _make_segsum(16)
_segsum128 = _make_segsum(128)


def _enc_body(x_ref, w_ref, b_ref, o_ref):
    o_ref[...] = (
        jnp.dot(x_ref[...], w_ref[...], preferred_element_type=jnp.float32)
        + b_ref[...]
    )


_enc = pl.pallas_call(
    _enc_body, out_shape=jax.ShapeDtypeStruct((_N, 16), jnp.float32))


def _bn_relu(y, g, b):
    mu = jnp.mean(y, axis=0, keepdims=True)
    var = jnp.mean((y - mu) ** 2, axis=0, keepdims=True)
    y = (y - mu) / jnp.sqrt(var + 1e-5) * g + b
    return jnp.maximum(y, 0.0)


def _layer0_body(part, z0, wl, bl, wr, g, b, zo, dego):
    agg = part[0, : _N] + part[1, : _N]
    deg = jnp.maximum(agg[:, 8:9], 1.0)
    mean = agg[:, 0:8] / deg
    z = z0[...][:, 0:8]
    y = (
        jnp.dot(mean, wl[...], preferred_element_type=jnp.float32)
        + bl[...]
        + jnp.dot(z, wr[...], preferred_element_type=jnp.float32)
    )
    zo[...] = _bn_relu(y, g[...], b[...])
    dego[...] = deg


_layer0 = pl.pallas_call(
    _layer0_body,
    out_shape=(
        jax.ShapeDtypeStruct((_N, 128), jnp.float32),
        jax.ShapeDtypeStruct((_N, 1), jnp.float32),
    ),
)


def _mid_body(part, z, deg, wl, bl, wr, g, b, zo):
    agg = part[0, : _N] + part[1, : _N]
    mean = agg / deg[...]
    y = (
        jnp.dot(mean, wl[...], preferred_element_type=jnp.float32)
        + bl[...]
        + jnp.dot(z[...], wr[...], preferred_element_type=jnp.float32)
    )
    zo[...] = _bn_relu(y, g[...], b[...])


_mid = pl.pallas_call(
    _mid_body, out_shape=jax.ShapeDtypeStruct((_N, 128), jnp.float32))


def _mid2_body(part, z, deg, wl, bl, wr, g, b, wl3p, zo, uo):
    agg = part[0, : _N] + part[1, : _N]
    mean = agg / deg[...]
    y = (
        jnp.dot(mean, wl[...], preferred_element_type=jnp.float32)
        + bl[...]
        + jnp.dot(z[...], wr[...], preferred_element_type=jnp.float32)
    )
    z3 = _bn_relu(y, g[...], b[...])
    zo[...] = z3
    uo[...] = jnp.dot(z3, wl3p[...], preferred_element_type=jnp.float32)


_mid2 = pl.pallas_call(
    _mid2_body,
    out_shape=(
        jax.ShapeDtypeStruct((_N, 128), jnp.float32),
        jax.ShapeDtypeStruct((_N, 16), jnp.float32),
    ),
)


def _fin_body(part, z3, deg, bl3, wr3, dw, db, ho):
    aggt = part[0, : _N, 0:8] + part[1, : _N, 0:8]
    meant = aggt / deg[...]
    z4 = (
        meant
        + bl3[...]
        + jnp.dot(z3[...], wr3[...], preferred_element_type=jnp.float32)
    )
    ho[...] = jnp.dot(z4, dw[...], preferred_element_type=jnp.float32) + db[...]


_fin = pl.pallas_call(
    _fin_body, out_shape=jax.ShapeDtypeStruct((_N, 4), jnp.float32))


def kernel(x, edge_index, params):
    p = params
    wencp = jnp.pad(p["enc_W"], ((0, 0), (0, 8)))
    bencp = jnp.pad(p["enc_b"], (0, 8)).at[8].set(1.0).reshape(1, 16)
    wl3p = jnp.pad(p["Wl"][3], ((0, 0), (0, 8)))
    bl = [b.reshape(1, -1) for b in p["bl"]]
    g = [v.reshape(1, -1) for v in p["bn_g"]]
    bb = [v.reshape(1, -1) for v in p["bn_b"]]
    db = p["dec_b"].reshape(1, 4)

    pad = _EP - _E
    srcg = jnp.concatenate(
        [edge_index[0], jnp.zeros((pad,), jnp.int32)]).reshape(_NC * _NS, _K, _CHUNK)
    dstg = jnp.concatenate(
        [edge_index[1], jnp.full((pad,), _N, jnp.int32)]).reshape(_NC * _NS, _K, _CHUNK)

    z0aug = _enc(x, wencp, bencp)
    part0 = _segsum16(z0aug, srcg, dstg)
    z1, deg = _layer0(part0, z0aug, p["Wl"][0], bl[0], p["Wr"][0], g[0], bb[0])
    part1 = _segsum128(z1, srcg, dstg)
    z2 = _mid(part1, z1, deg, p["Wl"][1], bl[1], p["Wr"][1], g[1], bb[1])
    part2 = _segsum128(z2, srcg, dstg)
    z3, u3p = _mid2(part2, z2, deg, p["Wl"][2], bl[2], p["Wr"][2], g[2], bb[2],
                    wl3p)
    part3 = _segsum16(u3p, srcg, dstg)
    h = _fin(part3, z3, deg, bl[3], p["Wr"][3], p["dec_W"], db)
    return h


# speedup vs baseline: 6.7278x; 1.0180x over previous
import functools

import jax
import jax.numpy as jnp
from jax import lax
from jax.experimental import pallas as pl
from jax.experimental.pallas import tpu as pltpu
from jax.experimental.pallas import tpu_sc as plsc

_N = 10000
_E = 320000
_NP = 10240
_NC = 2
_NS = 16
_L = 16
_CHUNK = 128
_K = 80
_EP = _NC * _NS * _K * _CHUNK
_RPT = _NP // _NS
_ZR = 64
_BS = 2
_HALF = _K // (2 * _BS)


def _make_segsum(D, feature_split):
    mesh = plsc.VectorSubcoreMesh(core_axis_name="c", subcore_axis_name="s")
    dc = D // 2 if feature_split else D
    k = _K * 2 if feature_split else _K
    half = k // (2 * _BS)
    zsteps = _ZR * (dc // _L)

    def body(z_hbm, srcg, dstg, out_hbm, src_v, dst_v, gbuf, zbuf, acc,
             gsa, gsb, ssa, ssb):
        c = lax.axis_index("c")
        s = lax.axis_index("s")
        w = s if feature_split else c * _NS + s
        zh = z_hbm.at[c] if feature_split else z_hbm

        def zb(i, carry):
            r = i // (dc // _L)
            col = (i % (dc // _L)) * _L
            zbuf[r, pl.ds(col, _L)] = jnp.zeros((_L,), jnp.float32)
            return carry

        lax.fori_loop(0, zsteps, zb, 0)

        def zacc(i, carry):
            pltpu.sync_copy(zbuf, acc.at[pl.ds(s * _RPT + i * _ZR, _ZR)])
            return carry

        lax.fori_loop(0, _RPT // _ZR, zacc, 0)

        pltpu.sync_copy(srcg.at[w], src_v)
        pltpu.sync_copy(dstg.at[w], dst_v)
        plsc.subcore_barrier()

        def fire_gather(step, base, sem):
            for b in range(_BS):
                pltpu.async_copy(
                    zh.at[src_v.at[step * _BS + b]], gbuf.at[base + b], sem)

        def wait_gather(step, base, sem):
            for b in range(_BS):
                pltpu.make_async_copy(
                    zh.at[src_v.at[step * _BS + b]], gbuf.at[base + b],
                    sem).wait()

        def fire_scatter(step, base, sem):
            for b in range(_BS):
                pltpu.async_copy(
                    gbuf.at[base + b], acc.at[dst_v.at[step * _BS + b]], sem,
                    add=True)

        def wait_scatter(step, base, sem):
            for b in range(_BS):
                pltpu.make_async_copy(
                    gbuf.at[base + b], acc.at[dst_v.at[step * _BS + b]],
                    sem).wait()

        fire_gather(0, 0, gsa)
        fire_gather(1, _BS, gsb)

        def step_fn(it, carry):
            s0 = 2 * it
            s1 = s0 + 1
            wait_gather(s0, 0, gsa)
            fire_scatter(s0, 0, ssa)
            wait_scatter(s0, 0, ssa)

            @pl.when(it + 1 < half)
            def _():
                fire_gather(s0 + 2, 0, gsa)

            wait_gather(s1, _BS, gsb)
            fire_scatter(s1, _BS, ssb)
            wait_scatter(s1, _BS, ssb)

            @pl.when(it + 1 < half)
            def _():
                fire_gather(s1 + 2, _BS, gsb)

            return carry

        lax.fori_loop(0, half, step_fn, 0)

        plsc.subcore_barrier()
        pltpu.sync_copy(acc.at[pl.ds(s * _RPT, _RPT)],
                        out_hbm.at[c, pl.ds(s * _RPT, _RPT)])

    return pl.kernel(
        body,
        out_type=jax.ShapeDtypeStruct((_NC, _NP, dc), jnp.float32),
        mesh=mesh,
        scratch_types=[
            pltpu.VMEM((k, _CHUNK), jnp.int32),
            pltpu.VMEM((k, _CHUNK), jnp.int32),
            pltpu.VMEM((2 * _BS, _CHUNK, dc), jnp.float32),
            pltpu.VMEM((_ZR, dc), jnp.float32),
            pltpu.VMEM_SHARED((_NP, dc), jnp.float32),
            pltpu.SemaphoreType.DMA,
            pltpu.SemaphoreType.DMA,
            pltpu.SemaphoreType.DMA,
            pltpu.SemaphoreType.DMA,
        ],
        compiler_params=pltpu.CompilerParams(use_tc_tiling_on_sc=False),
    )


_segsum16 = _make_segsum(16, feature_split=False)
_segsum128 = _make_segsum(128, feature_split=True)


def _enc_body(x_ref, w_ref, b_ref, o_ref):
    o_ref[...] = (
        jnp.dot(x_ref[...], w_ref[...], preferred_element_type=jnp.float32)
        + b_ref[...]
    )


_enc = pl.pallas_call(
    _enc_body, out_shape=jax.ShapeDtypeStruct((_N, 16), jnp.float32))


def _bn_relu(y, g, b):
    mu = jnp.mean(y, axis=0, keepdims=True)
    var = jnp.mean((y - mu) ** 2, axis=0, keepdims=True)
    y = (y - mu) / jnp.sqrt(var + 1e-5) * g + b
    return jnp.maximum(y, 0.0)


def _layer0_body(part, z0, wl, bl, wr, g, b, zo, dego):
    agg = part[0, : _N] + part[1, : _N]
    deg = jnp.maximum(agg[:, 8:9], 1.0)
    mean = agg[:, 0:8] / deg
    z = z0[...][:, 0:8]
    y = (
        jnp.dot(mean, wl[...], preferred_element_type=jnp.float32)
        + bl[...]
        + jnp.dot(z, wr[...], preferred_element_type=jnp.float32)
    )
    y = _bn_relu(y, g[...], b[...])
    zo[0] = y[:, 0:64]
    zo[1] = y[:, 64:128]
    dego[...] = deg


_layer0 = pl.pallas_call(
    _layer0_body,
    out_shape=(
        jax.ShapeDtypeStruct((2, _N, 64), jnp.float32),
        jax.ShapeDtypeStruct((_N, 1), jnp.float32),
    ),
)


def _mid_body(part, z, deg, wl, bl, wr, g, b, zo):
    agg = jnp.concatenate([part[0, : _N], part[1, : _N]], axis=1)
    mean = agg / deg[...]
    zc = jnp.concatenate([z[0], z[1]], axis=1)
    y = (
        jnp.dot(mean, wl[...], preferred_element_type=jnp.float32)
        + bl[...]
        + jnp.dot(zc, wr[...], preferred_element_type=jnp.float32)
    )
    y = _bn_relu(y, g[...], b[...])
    zo[0] = y[:, 0:64]
    zo[1] = y[:, 64:128]


_mid = pl.pallas_call(
    _mid_body, out_shape=jax.ShapeDtypeStruct((2, _N, 64), jnp.float32))


def _mid2_body(part, z, deg, wl, bl, wr, g, b, wl3p, zo, uo):
    agg = jnp.concatenate([part[0, : _N], part[1, : _N]], axis=1)
    mean = agg / deg[...]
    zc = jnp.concatenate([z[0], z[1]], axis=1)
    y = (
        jnp.dot(mean, wl[...], preferred_element_type=jnp.float32)
        + bl[...]
        + jnp.dot(zc, wr[...], preferred_element_type=jnp.float32)
    )
    z3 = _bn_relu(y, g[...], b[...])
    zo[...] = z3
    uo[...] = jnp.dot(z3, wl3p[...], preferred_element_type=jnp.float32)


_mid2 = pl.pallas_call(
    _mid2_body,
    out_shape=(
        jax.ShapeDtypeStruct((_N, 128), jnp.float32),
        jax.ShapeDtypeStruct((_N, 16), jnp.float32),
    ),
)


def _fin_body(part, z3, deg, bl3, wr3, dw, db, ho):
    aggt = part[0, : _N, 0:8] + part[1, : _N, 0:8]
    meant = aggt / deg[...]
    z4 = (
        meant
        + bl3[...]
        + jnp.dot(z3[...], wr3[...], preferred_element_type=jnp.float32)
    )
    ho[...] = jnp.dot(z4, dw[...], preferred_element_type=jnp.float32) + db[...]


_fin = pl.pallas_call(
    _fin_body, out_shape=jax.ShapeDtypeStruct((_N, 4), jnp.float32))


def kernel(x, edge_index, params):
    p = params
    wencp = jnp.pad(p["enc_W"], ((0, 0), (0, 8)))
    bencp = jnp.pad(p["enc_b"], (0, 8)).at[8].set(1.0).reshape(1, 16)
    wl3p = jnp.pad(p["Wl"][3], ((0, 0), (0, 8)))
    bl = [b.reshape(1, -1) for b in p["bl"]]
    g = [v.reshape(1, -1) for v in p["bn_g"]]
    bb = [v.reshape(1, -1) for v in p["bn_b"]]
    db = p["dec_b"].reshape(1, 4)

    pad = _EP - _E
    srcf = jnp.concatenate([edge_index[0], jnp.zeros((pad,), jnp.int32)])
    dstf = jnp.concatenate([edge_index[1], jnp.full((pad,), _N, jnp.int32)])
    srcg_e = srcf.reshape(_NC * _NS, _K, _CHUNK)
    dstg_e = dstf.reshape(_NC * _NS, _K, _CHUNK)
    srcg_f = srcf.reshape(_NS, 2 * _K, _CHUNK)
    dstg_f = dstf.reshape(_NS, 2 * _K, _CHUNK)

    z0aug = _enc(x, wencp, bencp)
    part0 = _segsum16(z0aug, srcg_e, dstg_e)
    z1, deg = _layer0(part0, z0aug, p["Wl"][0], bl[0], p["Wr"][0], g[0], bb[0])
    part1 = _segsum128(z1, srcg_f, dstg_f)
    z2 = _mid(part1, z1, deg, p["Wl"][1], bl[1], p["Wr"][1], g[1], bb[1])
    part2 = _segsum128(z2, srcg_f, dstg_f)
    z3, u3p = _mid2(part2, z2, deg, p["Wl"][2], bl[2], p["Wr"][2], g[2], bb[2],
                    wl3p)
    part3 = _segsum16(u3p, srcg_e, dstg_e)
    h = _fin(part3, z3, deg, bl[3], p["Wr"][3], p["dec_W"], db)
    return h


# speedup vs baseline: 7.1860x; 1.0681x over previous
import functools

import jax
import jax.numpy as jnp
from jax import lax
from jax.experimental import pallas as pl
from jax.experimental.pallas import tpu as pltpu
from jax.experimental.pallas import tpu_sc as plsc

_N = 10000
_E = 320000
_NP = 10240
_NC = 2
_NS = 16
_CHUNK = 128
_K = 80
_EP = _NC * _NS * _K * _CHUNK
_RPT = _NP // _NS
_ZR = 64
_BS = 2


def _make_segsum(D, dtype):
    mesh = plsc.VectorSubcoreMesh(core_axis_name="c", subcore_axis_name="s")
    lanes = 32 if dtype == jnp.bfloat16 else 16
    half = _K // (2 * _BS)
    zsteps = _ZR * (D // lanes)

    def body(z_hbm, srcg, dstg, out_hbm, src_v, dst_v, gbuf, zbuf, acc,
             gsa, gsb, ssa, ssb):
        c = lax.axis_index("c")
        s = lax.axis_index("s")
        w = c * _NS + s

        def zb(i, carry):
            r = i // (D // lanes)
            col = (i % (D // lanes)) * lanes
            zbuf[r, pl.ds(col, lanes)] = jnp.zeros((lanes,), dtype)
            return carry

        lax.fori_loop(0, zsteps, zb, 0)

        def zacc(i, carry):
            pltpu.sync_copy(zbuf, acc.at[pl.ds(s * _RPT + i * _ZR, _ZR)])
            return carry

        lax.fori_loop(0, _RPT // _ZR, zacc, 0)

        pltpu.sync_copy(srcg.at[w], src_v)
        pltpu.sync_copy(dstg.at[w], dst_v)
        plsc.subcore_barrier()

        def fire_gather(step, base, sem):
            for b in range(_BS):
                pltpu.async_copy(
                    z_hbm.at[src_v.at[step * _BS + b]], gbuf.at[base + b], sem)

        def wait_gather(step, base, sem):
            for b in range(_BS):
                pltpu.make_async_copy(
                    z_hbm.at[src_v.at[step * _BS + b]], gbuf.at[base + b],
                    sem).wait()

        def fire_scatter(step, base, sem):
            for b in range(_BS):
                pltpu.async_copy(
                    gbuf.at[base + b], acc.at[dst_v.at[step * _BS + b]], sem,
                    add=True)

        def wait_scatter(step, base, sem):
            for b in range(_BS):
                pltpu.make_async_copy(
                    gbuf.at[base + b], acc.at[dst_v.at[step * _BS + b]],
                    sem).wait()

        fire_gather(0, 0, gsa)
        fire_gather(1, _BS, gsb)

        def step_fn(it, carry):
            s0 = 2 * it
            s1 = s0 + 1
            wait_gather(s0, 0, gsa)
            fire_scatter(s0, 0, ssa)
            wait_scatter(s0, 0, ssa)

            @pl.when(it + 1 < half)
            def _():
                fire_gather(s0 + 2, 0, gsa)

            wait_gather(s1, _BS, gsb)
            fire_scatter(s1, _BS, ssb)
            wait_scatter(s1, _BS, ssb)

            @pl.when(it + 1 < half)
            def _():
                fire_gather(s1 + 2, _BS, gsb)

            return carry

        lax.fori_loop(0, half, step_fn, 0)

        plsc.subcore_barrier()
        pltpu.sync_copy(acc.at[pl.ds(s * _RPT, _RPT)],
                        out_hbm.at[c, pl.ds(s * _RPT, _RPT)])

    return pl.kernel(
        body,
        out_type=jax.ShapeDtypeStruct((_NC, _NP, D), dtype),
        mesh=mesh,
        scratch_types=[
            pltpu.VMEM((_K, _CHUNK), jnp.int32),
            pltpu.VMEM((_K, _CHUNK), jnp.int32),
            pltpu.VMEM((2 * _BS, _CHUNK, D), dtype),
            pltpu.VMEM((_ZR, D), dtype),
            pltpu.VMEM_SHARED((_NP, D), dtype),
            pltpu.SemaphoreType.DMA,
            pltpu.SemaphoreType.DMA,
            pltpu.SemaphoreType.DMA,
            pltpu.SemaphoreType.DMA,
        ],
        compiler_params=pltpu.CompilerParams(use_tc_tiling_on_sc=False),
    )


_segsum16 = _make_segsum(16, jnp.float32)
_segsum128 = _make_segsum(128, jnp.bfloat16)


def _enc_body(x_ref, w_ref, b_ref, o_ref):
    o_ref[...] = (
        jnp.dot(x_ref[...], w_ref[...], preferred_element_type=jnp.float32)
        + b_ref[...]
    )


_enc = pl.pallas_call(
    _enc_body, out_shape=jax.ShapeDtypeStruct((_N, 16), jnp.float32))


def _bn_relu(y, g, b):
    mu = jnp.mean(y, axis=0, keepdims=True)
    var = jnp.mean((y - mu) ** 2, axis=0, keepdims=True)
    y = (y - mu) / jnp.sqrt(var + 1e-5) * g + b
    return jnp.maximum(y, 0.0)


def _layer0_body(part, z0, wl, bl, wr, g, b, zo, zbo, dego):
    agg = part[0, : _N] + part[1, : _N]
    deg = jnp.maximum(agg[:, 8:9], 1.0)
    mean = agg[:, 0:8] / deg
    z = z0[...][:, 0:8]
    y = (
        jnp.dot(mean, wl[...], preferred_element_type=jnp.float32)
        + bl[...]
        + jnp.dot(z, wr[...], preferred_element_type=jnp.float32)
    )
    y = _bn_relu(y, g[...], b[...])
    zo[...] = y
    zbo[...] = y.astype(jnp.bfloat16)
    dego[...] = deg


_layer0 = pl.pallas_call(
    _layer0_body,
    out_shape=(
        jax.ShapeDtypeStruct((_N, 128), jnp.float32),
        jax.ShapeDtypeStruct((_N, 128), jnp.bfloat16),
        jax.ShapeDtypeStruct((_N, 1), jnp.float32),
    ),
)


def _mid_body(part, z, deg, wl, bl, wr, g, b, zo, zbo):
    agg = (part[0, : _N].astype(jnp.float32)
           + part[1, : _N].astype(jnp.float32))
    mean = agg / deg[...]
    y = (
        jnp.dot(mean, wl[...], preferred_element_type=jnp.float32)
        + bl[...]
        + jnp.dot(z[...], wr[...], preferred_element_type=jnp.float32)
    )
    y = _bn_relu(y, g[...], b[...])
    zo[...] = y
    zbo[...] = y.astype(jnp.bfloat16)


_mid = pl.pallas_call(
    _mid_body,
    out_shape=(
        jax.ShapeDtypeStruct((_N, 128), jnp.float32),
        jax.ShapeDtypeStruct((_N, 128), jnp.bfloat16),
    ),
)


def _mid2_body(part, z, deg, wl, bl, wr, g, b, wl3p, zo, uo):
    agg = (part[0, : _N].astype(jnp.float32)
           + part[1, : _N].astype(jnp.float32))
    mean = agg / deg[...]
    y = (
        jnp.dot(mean, wl[...], preferred_element_type=jnp.float32)
        + bl[...]
        + jnp.dot(z[...], wr[...], preferred_element_type=jnp.float32)
    )
    z3 = _bn_relu(y, g[...], b[...])
    zo[...] = z3
    uo[...] = jnp.dot(z3, wl3p[...], preferred_element_type=jnp.float32)


_mid2 = pl.pallas_call(
    _mid2_body,
    out_shape=(
        jax.ShapeDtypeStruct((_N, 128), jnp.float32),
        jax.ShapeDtypeStruct((_N, 16), jnp.float32),
    ),
)


def _fin_body(part, z3, deg, bl3, wr3, dw, db, ho):
    aggt = part[0, : _N, 0:8] + part[1, : _N, 0:8]
    meant = aggt / deg[...]
    z4 = (
        meant
        + bl3[...]
        + jnp.dot(z3[...], wr3[...], preferred_element_type=jnp.float32)
    )
    ho[...] = jnp.dot(z4, dw[...], preferred_element_type=jnp.float32) + db[...]


_fin = pl.pallas_call(
    _fin_body, out_shape=jax.ShapeDtypeStruct((_N, 4), jnp.float32))


def kernel(x, edge_index, params):
    p = params
    wencp = jnp.pad(p["enc_W"], ((0, 0), (0, 8)))
    bencp = jnp.pad(p["enc_b"], (0, 8)).at[8].set(1.0).reshape(1, 16)
    wl3p = jnp.pad(p["Wl"][3], ((0, 0), (0, 8)))
    bl = [b.reshape(1, -1) for b in p["bl"]]
    g = [v.reshape(1, -1) for v in p["bn_g"]]
    bb = [v.reshape(1, -1) for v in p["bn_b"]]
    db = p["dec_b"].reshape(1, 4)

    pad = _EP - _E
    srcg = jnp.concatenate(
        [edge_index[0], jnp.zeros((pad,), jnp.int32)]).reshape(_NC * _NS, _K, _CHUNK)
    dstg = jnp.concatenate(
        [edge_index[1], jnp.full((pad,), _N, jnp.int32)]).reshape(_NC * _NS, _K, _CHUNK)

    z0aug = _enc(x, wencp, bencp)
    part0 = _segsum16(z0aug, srcg, dstg)
    z1, z1b, deg = _layer0(part0, z0aug, p["Wl"][0], bl[0], p["Wr"][0],
                           g[0], bb[0])
    part1 = _segsum128(z1b, srcg, dstg)
    z2, z2b = _mid(part1, z1, deg, p["Wl"][1], bl[1], p["Wr"][1], g[1], bb[1])
    part2 = _segsum128(z2b, srcg, dstg)
    z3, u3p = _mid2(part2, z2, deg, p["Wl"][2], bl[2], p["Wr"][2], g[2], bb[2],
                    wl3p)
    part3 = _segsum16(u3p, srcg, dstg)
    h = _fin(part3, z3, deg, bl[3], p["Wr"][3], p["dec_W"], db)
    return h


# speedup vs baseline: 7.5233x; 1.0469x over previous
import functools

import jax
import jax.numpy as jnp
from jax import lax
from jax.experimental import pallas as pl
from jax.experimental.pallas import tpu as pltpu
from jax.experimental.pallas import tpu_sc as plsc

_N = 10000
_E = 320000
_NP = 10240
_NC = 2
_NS = 16
_CHUNK = 128
_K = 80
_EP = _NC * _NS * _K * _CHUNK
_RPT = _NP // _NS
_ZR = 64
_BS = 2


def _make_segsum(D, dtype, k0, k1):
    assert (k0 + k1) * _NS * _CHUNK == _EP
    assert k0 % (2 * _BS) == 0 and k1 % (2 * _BS) == 0
    mesh = plsc.VectorSubcoreMesh(core_axis_name="c", subcore_axis_name="s")
    lanes = 32 if dtype == jnp.bfloat16 else 16
    zsteps = _ZR * (D // lanes)

    def body(z_hbm, srcg, dstg, out_hbm, src_v, dst_v, gbuf, zbuf, acc,
             gsa, gsb, ssa, ssb):
        c = lax.axis_index("c")
        s = lax.axis_index("s")

        def zb(i, carry):
            r = i // (D // lanes)
            col = (i % (D // lanes)) * lanes
            zbuf[r, pl.ds(col, lanes)] = jnp.zeros((lanes,), dtype)
            return carry

        lax.fori_loop(0, zsteps, zb, 0)

        def zacc(i, carry):
            pltpu.sync_copy(zbuf, acc.at[pl.ds(s * _RPT + i * _ZR, _ZR)])
            return carry

        lax.fori_loop(0, _RPT // _ZR, zacc, 0)

        def run_pipeline(kc, row0):
            pltpu.sync_copy(srcg.at[pl.ds(row0, kc)], src_v.at[pl.ds(0, kc)])
            pltpu.sync_copy(dstg.at[pl.ds(row0, kc)], dst_v.at[pl.ds(0, kc)])
            half = kc // (2 * _BS)

            def fire_gather(step, base, sem):
                for b in range(_BS):
                    pltpu.async_copy(
                        z_hbm.at[src_v.at[step * _BS + b]], gbuf.at[base + b],
                        sem)

            def wait_gather(step, base, sem):
                for b in range(_BS):
                    pltpu.make_async_copy(
                        z_hbm.at[src_v.at[step * _BS + b]], gbuf.at[base + b],
                        sem).wait()

            def fire_scatter(step, base, sem):
                for b in range(_BS):
                    pltpu.async_copy(
                        gbuf.at[base + b], acc.at[dst_v.at[step * _BS + b]],
                        sem, add=True)

            def wait_scatter(step, base, sem):
                for b in range(_BS):
                    pltpu.make_async_copy(
                        gbuf.at[base + b], acc.at[dst_v.at[step * _BS + b]],
                        sem).wait()

            fire_gather(0, 0, gsa)
            fire_gather(1, _BS, gsb)

            def step_fn(it, carry):
                s0 = 2 * it
                s1 = s0 + 1
                wait_gather(s0, 0, gsa)
                fire_scatter(s0, 0, ssa)
                wait_scatter(s0, 0, ssa)

                @pl.when(it + 1 < half)
                def _():
                    fire_gather(s0 + 2, 0, gsa)

                wait_gather(s1, _BS, gsb)
                fire_scatter(s1, _BS, ssb)
                wait_scatter(s1, _BS, ssb)

                @pl.when(it + 1 < half)
                def _():
                    fire_gather(s1 + 2, _BS, gsb)

                return carry

            lax.fori_loop(0, half, step_fn, 0)

        plsc.subcore_barrier()

        @pl.when(c == 0)
        def _():
            run_pipeline(k0, s * k0)

        @pl.when(c == 1)
        def _():
            run_pipeline(k1, _NS * k0 + s * k1)

        plsc.subcore_barrier()
        pltpu.sync_copy(acc.at[pl.ds(s * _RPT, _RPT)],
                        out_hbm.at[c, pl.ds(s * _RPT, _RPT)])

    return pl.kernel(
        body,
        out_type=jax.ShapeDtypeStruct((_NC, _NP, D), dtype),
        mesh=mesh,
        scratch_types=[
            pltpu.VMEM((max(k0, k1), _CHUNK), jnp.int32),
            pltpu.VMEM((max(k0, k1), _CHUNK), jnp.int32),
            pltpu.VMEM((2 * _BS, _CHUNK, D), dtype),
            pltpu.VMEM((_ZR, D), dtype),
            pltpu.VMEM_SHARED((_NP, D), dtype),
            pltpu.SemaphoreType.DMA,
            pltpu.SemaphoreType.DMA,
            pltpu.SemaphoreType.DMA,
            pltpu.SemaphoreType.DMA,
        ],
        compiler_params=pltpu.CompilerParams(use_tc_tiling_on_sc=False),
    )


_segsum16 = _make_segsum(16, jnp.float32, 96, 64)
_segsum128 = _make_segsum(128, jnp.bfloat16, 128, 32)


def _enc_body(x_ref, w_ref, b_ref, o_ref):
    o_ref[...] = (
        jnp.dot(x_ref[...], w_ref[...], preferred_element_type=jnp.float32)
        + b_ref[...]
    )


_enc = pl.pallas_call(
    _enc_body, out_shape=jax.ShapeDtypeStruct((_N, 16), jnp.float32))


def _bn_relu(y, g, b):
    mu = jnp.mean(y, axis=0, keepdims=True)
    var = jnp.mean((y - mu) ** 2, axis=0, keepdims=True)
    y = (y - mu) / jnp.sqrt(var + 1e-5) * g + b
    return jnp.maximum(y, 0.0)


def _layer0_body(part, z0, wl, bl, wr, g, b, zo, zbo, dego):
    agg = part[0, : _N] + part[1, : _N]
    deg = jnp.maximum(agg[:, 8:9], 1.0)
    mean = agg[:, 0:8] / deg
    z = z0[...][:, 0:8]
    y = (
        jnp.dot(mean, wl[...], preferred_element_type=jnp.float32)
        + bl[...]
        + jnp.dot(z, wr[...], preferred_element_type=jnp.float32)
    )
    y = _bn_relu(y, g[...], b[...])
    zo[...] = y
    zbo[...] = y.astype(jnp.bfloat16)
    dego[...] = deg


_layer0 = pl.pallas_call(
    _layer0_body,
    out_shape=(
        jax.ShapeDtypeStruct((_N, 128), jnp.float32),
        jax.ShapeDtypeStruct((_N, 128), jnp.bfloat16),
        jax.ShapeDtypeStruct((_N, 1), jnp.float32),
    ),
)


def _mid_body(part, z, deg, wl, bl, wr, g, b, zo, zbo):
    agg = (part[0, : _N].astype(jnp.float32)
           + part[1, : _N].astype(jnp.float32))
    mean = agg / deg[...]
    y = (
        jnp.dot(mean, wl[...], preferred_element_type=jnp.float32)
        + bl[...]
        + jnp.dot(z[...], wr[...], preferred_element_type=jnp.float32)
    )
    y = _bn_relu(y, g[...], b[...])
    zo[...] = y
    zbo[...] = y.astype(jnp.bfloat16)


_mid = pl.pallas_call(
    _mid_body,
    out_shape=(
        jax.ShapeDtypeStruct((_N, 128), jnp.float32),
        jax.ShapeDtypeStruct((_N, 128), jnp.bfloat16),
    ),
)


def _mid2_body(part, z, deg, wl, bl, wr, g, b, wl3p, zo, uo):
    agg = (part[0, : _N].astype(jnp.float32)
           + part[1, : _N].astype(jnp.float32))
    mean = agg / deg[...]
    y = (
        jnp.dot(mean, wl[...], preferred_element_type=jnp.float32)
        + bl[...]
        + jnp.dot(z[...], wr[...], preferred_element_type=jnp.float32)
    )
    z3 = _bn_relu(y, g[...], b[...])
    zo[...] = z3
    uo[...] = jnp.dot(z3, wl3p[...], preferred_element_type=jnp.float32)


_mid2 = pl.pallas_call(
    _mid2_body,
    out_shape=(
        jax.ShapeDtypeStruct((_N, 128), jnp.float32),
        jax.ShapeDtypeStruct((_N, 16), jnp.float32),
    ),
)


def _fin_body(part, z3, deg, bl3, wr3, dw, db, ho):
    aggt = part[0, : _N, 0:8] + part[1, : _N, 0:8]
    meant = aggt / deg[...]
    z4 = (
        meant
        + bl3[...]
        + jnp.dot(z3[...], wr3[...], preferred_element_type=jnp.float32)
    )
    ho[...] = jnp.dot(z4, dw[...], preferred_element_type=jnp.float32) + db[...]


_fin = pl.pallas_call(
    _fin_body, out_shape=jax.ShapeDtypeStruct((_N, 4), jnp.float32))


def kernel(x, edge_index, params):
    p = params
    wencp = jnp.pad(p["enc_W"], ((0, 0), (0, 8)))
    bencp = jnp.pad(p["enc_b"], (0, 8)).at[8].set(1.0).reshape(1, 16)
    wl3p = jnp.pad(p["Wl"][3], ((0, 0), (0, 8)))
    bl = [b.reshape(1, -1) for b in p["bl"]]
    g = [v.reshape(1, -1) for v in p["bn_g"]]
    bb = [v.reshape(1, -1) for v in p["bn_b"]]
    db = p["dec_b"].reshape(1, 4)

    pad = _EP - _E
    srcg = jnp.concatenate(
        [edge_index[0], jnp.zeros((pad,), jnp.int32)]).reshape(-1, _CHUNK)
    dstg = jnp.concatenate(
        [edge_index[1], jnp.full((pad,), _N, jnp.int32)]).reshape(-1, _CHUNK)

    z0aug = _enc(x, wencp, bencp)
    part0 = _segsum16(z0aug, srcg, dstg)
    z1, z1b, deg = _layer0(part0, z0aug, p["Wl"][0], bl[0], p["Wr"][0],
                           g[0], bb[0])
    part1 = _segsum128(z1b, srcg, dstg)
    z2, z2b = _mid(part1, z1, deg, p["Wl"][1], bl[1], p["Wr"][1], g[1], bb[1])
    part2 = _segsum128(z2b, srcg, dstg)
    z3, u3p = _mid2(part2, z2, deg, p["Wl"][2], bl[2], p["Wr"][2], g[2], bb[2],
                    wl3p)
    part3 = _segsum16(u3p, srcg, dstg)
    h = _fin(part3, z3, deg, bl[3], p["Wr"][3], p["dec_W"], db)
    return h
